# Initial kernel scaffold; baseline (speedup 1.0000x reference)
import functools

import jax
import jax.numpy as jnp
from jax import lax
from jax.experimental import pallas as pl
from jax.experimental.pallas import tpu as pltpu
from jax.experimental.pallas import tpu_sc as plsc

_NC = 2
_NS = 16
_NW = _NC * _NS
_CH = 128
_LANES = 16


def _make_aggregate(N, D, E, compute_deg):
    assert E % _CH == 0 and D % _LANES == 0 and N % _NS == 0
    n_chunks = E // _CH
    nb, rem = divmod(n_chunks, _NW)
    rows_per_tile = N // _NS
    zp = ((N + _NW - 1) // _NW + 7) // 8 * 8
    npad = _NW * zp
    last_t = N // zp
    last_sz = N - last_t * zp

    mesh = plsc.VectorSubcoreMesh(core_axis_name="c", subcore_axis_name="s")

    out_type = [jax.ShapeDtypeStruct((_NC, N, D), jnp.float32)]
    if compute_deg:
        out_type.append(jax.ShapeDtypeStruct((_NC, N), jnp.float32))

    scratch = [
        pltpu.VMEM((_CH,), jnp.int32),
        pltpu.VMEM((_CH,), jnp.int32),
        pltpu.VMEM((_CH,), jnp.float32),
        pltpu.VMEM((_CH, D), jnp.float32),
        pltpu.VMEM_SHARED((N, D), jnp.float32),
        pltpu.VMEM_SHARED((npad,), jnp.float32),
        pltpu.SemaphoreType.DMA,
    ]

    def body(x_hbm, src_hbm, dst_hbm, w_hbm, *refs):
        if compute_deg:
            out_acc, out_deg = refs[0], refs[1]
            refs = refs[2:]
        else:
            out_acc = refs[0]
            refs = refs[1:]
        src_v, dst_v, w_v, rows_v, acc, deg_acc, sem = refs

        c = lax.axis_index("c")
        s = lax.axis_index("s")
        wid = s * _NC + c

        def zrow(r, carry):
            for cb in range(D // _LANES):
                rows_v[r, pl.ds(cb * _LANES, _LANES)] = jnp.zeros(
                    (_LANES,), jnp.float32)
            return carry
        lax.fori_loop(0, _CH, zrow, 0)
        for g in range(_CH // _LANES):
            w_v[pl.ds(g * _LANES, _LANES)] = jnp.zeros((_LANES,), jnp.float32)

        r0 = s * rows_per_tile
        full, tail = divmod(rows_per_tile, _CH)
        for i in range(full):
            pltpu.sync_copy(rows_v, acc.at[pl.ds(r0 + i * _CH, _CH)])
        if tail:
            pltpu.sync_copy(rows_v.at[pl.ds(0, tail)],
                            acc.at[pl.ds(r0 + full * _CH, tail)])
        dz0 = s * (npad // _NS)
        dfull, dtail = divmod(npad // _NS, _CH)
        for i in range(dfull):
            pltpu.sync_copy(w_v, deg_acc.at[pl.ds(dz0 + i * _CH, _CH)])
        if dtail:
            pltpu.sync_copy(w_v.at[pl.ds(0, dtail)],
                            deg_acc.at[pl.ds(dz0 + dfull * _CH, dtail)])
        plsc.subcore_barrier()

        def edge_chunk(k, carry):
            base = (wid + k * _NW) * _CH
            pltpu.sync_copy(src_hbm.at[pl.ds(base, _CH)], src_v)
            pltpu.sync_copy(dst_hbm.at[pl.ds(base, _CH)], dst_v)
            pltpu.sync_copy(w_hbm.at[pl.ds(base, _CH)], w_v)
            pltpu.async_copy(x_hbm.at[src_v], rows_v, sem).wait()

            def scale(r, cc):
                wr = w_v[r]
                for cb in range(D // _LANES):
                    sl = pl.ds(cb * _LANES, _LANES)
                    rows_v[r, sl] = rows_v[r, sl] * wr
                return cc
            lax.fori_loop(0, _CH, scale, 0)

            pltpu.sync_copy(rows_v, acc.at[dst_v], add=True)
            if compute_deg:
                pltpu.sync_copy(w_v, deg_acc.at[dst_v], add=True)
            return carry

        n_mine = nb + jnp.where(wid < rem, 1, 0)
        lax.fori_loop(0, n_mine, edge_chunk, 0)
        plsc.subcore_barrier()

        pltpu.sync_copy(acc.at[pl.ds(r0, rows_per_tile)],
                        out_acc.at[c, pl.ds(r0, rows_per_tile)])
        if compute_deg:
            db = wid * zp

            @pl.when(wid < last_t)
            def _():
                pltpu.sync_copy(deg_acc.at[pl.ds(db, zp)],
                                out_deg.at[c, pl.ds(db, zp)])
            if last_sz:
                @pl.when(wid == last_t)
                def _():
                    pltpu.sync_copy(deg_acc.at[pl.ds(last_t * zp, last_sz)],
                                    out_deg.at[c, pl.ds(last_t * zp, last_sz)])

    return pl.kernel(body, out_type, mesh=mesh, scratch_types=scratch)


def _tc_mlp1(acc_ref, d0_ref, d1_ref, w_ref, b_ref, out_ref):
    x = acc_ref[0] + acc_ref[1]
    deg = jnp.maximum(d0_ref[...] + d1_ref[...], 1.0)
    h = jnp.dot(x / deg, w_ref[...], preferred_element_type=jnp.float32)
    out_ref[...] = jnp.maximum(h + b_ref[...], 0.0)


def _tc_mlp2(acc_ref, d0_ref, d1_ref, wemb_ref, bemb_ref, wcls_ref, bcls_ref,
             emb_ref, log_ref):
    x = acc_ref[0] + acc_ref[1]
    deg = jnp.maximum(d0_ref[...] + d1_ref[...], 1.0)
    e = jnp.dot(x / deg, wemb_ref[...],
                preferred_element_type=jnp.float32) + bemb_ref[...]
    emb_ref[...] = e
    log_ref[...] = jnp.dot(e, wcls_ref[...],
                           preferred_element_type=jnp.float32) + bcls_ref[...]


def kernel(node_features, edge_index, edge_weight, W_in, b_in, W_emb, b_emb,
           W_cls, b_cls):
    N, D = node_features.shape
    H = W_in.shape[1]
    EMB = W_emb.shape[1]
    C = W_cls.shape[1]
    E = edge_weight.shape[0]
    src = edge_index[0]
    dst = edge_index[1]

    acc1, deg = _make_aggregate(N, D, E, True)(
        node_features, src, dst, edge_weight)
    d0 = deg[0][:, None]
    d1 = deg[1][:, None]

    rb = 1000 if N % 1000 == 0 else N
    grid = (N // rb,)
    hidden = pl.pallas_call(
        _tc_mlp1,
        grid=grid,
        in_specs=[
            pl.BlockSpec((_NC, rb, D), lambda i: (0, i, 0)),
            pl.BlockSpec((rb, 1), lambda i: (i, 0)),
            pl.BlockSpec((rb, 1), lambda i: (i, 0)),
            pl.BlockSpec((D, H), lambda i: (0, 0)),
            pl.BlockSpec((1, H), lambda i: (0, 0)),
        ],
        out_specs=pl.BlockSpec((rb, H), lambda i: (i, 0)),
        out_shape=jax.ShapeDtypeStruct((N, H), jnp.float32),
    )(acc1, d0, d1, W_in, b_in.reshape(1, H))

    (acc2,) = _make_aggregate(N, H, E, False)(hidden, src, dst, edge_weight)

    embedding, logits = pl.pallas_call(
        _tc_mlp2,
        grid=grid,
        in_specs=[
            pl.BlockSpec((_NC, rb, H), lambda i: (0, i, 0)),
            pl.BlockSpec((rb, 1), lambda i: (i, 0)),
            pl.BlockSpec((rb, 1), lambda i: (i, 0)),
            pl.BlockSpec((H, EMB), lambda i: (0, 0)),
            pl.BlockSpec((1, EMB), lambda i: (0, 0)),
            pl.BlockSpec((EMB, C), lambda i: (0, 0)),
            pl.BlockSpec((1, C), lambda i: (0, 0)),
        ],
        out_specs=[
            pl.BlockSpec((rb, EMB), lambda i: (i, 0)),
            pl.BlockSpec((rb, C), lambda i: (i, 0)),
        ],
        out_shape=[
            jax.ShapeDtypeStruct((N, EMB), jnp.float32),
            jax.ShapeDtypeStruct((N, C), jnp.float32),
        ],
    )(acc2, d0, d1, W_emb, b_emb.reshape(1, EMB), W_cls, b_cls.reshape(1, C))

    return (embedding, logits)


# speedup vs baseline: 5.3800x; 5.3800x over previous
import functools

import jax
import jax.numpy as jnp
from jax import lax
from jax.experimental import pallas as pl
from jax.experimental.pallas import tpu as pltpu
from jax.experimental.pallas import tpu_sc as plsc

_NC = 2
_NS = 16
_NW = _NC * _NS
_CH = 128
_LANES = 16


def _make_aggregate(N, D, E, compute_deg):
    assert E % _CH == 0 and D % _LANES == 0 and N % _NS == 0
    n_chunks = E // _CH
    nb, rem = divmod(n_chunks, _NW)
    rp8 = ((N + _NS - 1) // _NS + 7) // 8 * 8
    n_full = N // rp8
    tail_rows = N - n_full * rp8
    dzp = (((N + _NS - 1) // _NS) + 127) // 128 * 128
    npad = _NS * dzp

    mesh = plsc.VectorSubcoreMesh(core_axis_name="c", subcore_axis_name="s")

    out_type = [jax.ShapeDtypeStruct((_NC, N, D), jnp.float32)]
    if compute_deg:
        out_type.append(jax.ShapeDtypeStruct((_NC * npad,), jnp.float32))

    nbmax = nb + (1 if rem else 0)
    npc = 3 * ((nbmax + 2) // 3)

    scratch = (
        [pltpu.VMEM((_CH,), jnp.int32) for _ in range(3)]
        + [pltpu.VMEM((_CH,), jnp.int32) for _ in range(3)]
        + [pltpu.VMEM((_CH,), jnp.float32) for _ in range(3)]
        + [pltpu.VMEM((_CH, D), jnp.float32) for _ in range(3)]
        + [
            pltpu.VMEM_SHARED((N, D), jnp.float32),
            pltpu.VMEM_SHARED((npad,), jnp.float32),
        ]
        + [pltpu.SemaphoreType.DMA for _ in range(9)]
    )

    def body(x_hbm, src_hbm, dst_hbm, w_hbm, *refs):
        if compute_deg:
            out_acc, out_deg = refs[0], refs[1]
            refs = refs[2:]
        else:
            out_acc = refs[0]
            refs = refs[1:]
        src_s = refs[0:3]
        dst_s = refs[3:6]
        w_s = refs[6:9]
        rows_s = refs[9:12]
        acc, deg_acc = refs[12], refs[13]
        gsem = refs[14:17]
        ssem = refs[17:20]
        dsem = refs[20:23]
        rows_v = rows_s[0]
        w_v = w_s[0]

        c = lax.axis_index("c")
        s = lax.axis_index("s")
        wid = s * _NC + c

        def zrow(r, carry):
            for cb in range(D // _LANES):
                rows_v[r, pl.ds(cb * _LANES, _LANES)] = jnp.zeros(
                    (_LANES,), jnp.float32)
            return carry
        lax.fori_loop(0, _CH, zrow, 0)
        for g in range(_CH // _LANES):
            w_v[pl.ds(g * _LANES, _LANES)] = jnp.zeros((_LANES,), jnp.float32)

        r0 = s * rp8

        def _zero_rows(nrows):
            full, tail = divmod(nrows, _CH)
            for i in range(full):
                pltpu.sync_copy(rows_v, acc.at[pl.ds(r0 + i * _CH, _CH)])
            if tail:
                pltpu.sync_copy(rows_v.at[pl.ds(0, tail)],
                                acc.at[pl.ds(r0 + full * _CH, tail)])

        @pl.when(s < n_full)
        def _():
            _zero_rows(rp8)
        if tail_rows:
            @pl.when(s == n_full)
            def _():
                _zero_rows(tail_rows)

        dz0 = s * dzp
        dfull, dtail = divmod(dzp, _CH)
        for i in range(dfull):
            pltpu.sync_copy(w_v, deg_acc.at[pl.ds(dz0 + i * _CH, _CH)])
        if dtail:
            pltpu.sync_copy(w_v.at[pl.ds(0, dtail)],
                            deg_acc.at[pl.ds(dz0 + dfull * _CH, dtail)])
        plsc.subcore_barrier()

        n_real = nb + jnp.where(wid < rem, 1, 0)

        def chunk_base(m):
            return (wid + lax.rem(m, n_real) * _NW) * _CH

        def load_idx(m, s):
            base = chunk_base(m)
            pltpu.sync_copy(src_hbm.at[pl.ds(base, _CH)], src_s[s])
            pltpu.sync_copy(dst_hbm.at[pl.ds(base, _CH)], dst_s[s])
            pltpu.sync_copy(w_hbm.at[pl.ds(base, _CH)], w_s[s])

        def start_gather(s):
            pltpu.async_copy(x_hbm.at[src_s[s]], rows_s[s], gsem[s])

        def scale_slot(s, valid):
            def scale(g, cc):
                w_grp = w_s[s][pl.ds(g * _LANES, _LANES)] * valid
                w_s[s][pl.ds(g * _LANES, _LANES)] = w_grp
                for j in range(_LANES):
                    wr = w_grp[j]
                    r = g * _LANES + j
                    for cb in range(D // _LANES):
                        sl = pl.ds(cb * _LANES, _LANES)
                        rows_s[s][r, sl] = rows_s[s][r, sl] * wr
                return cc
            lax.fori_loop(0, _CH // _LANES, scale, 0)

        def start_scatter(s):
            pltpu.async_copy(rows_s[s], acc.at[dst_s[s]], ssem[s], add=True)
            if compute_deg:
                pltpu.async_copy(w_s[s], deg_acc.at[dst_s[s]], dsem[s],
                                 add=True)

        def drain_scatter(s):
            pltpu.make_async_copy(rows_s[s], acc.at[dst_s[s]], ssem[s]).wait()
            if compute_deg:
                pltpu.make_async_copy(w_s[s], deg_acc.at[dst_s[s]],
                                      dsem[s]).wait()

        def chunk_body(m, carry):
            load_idx(m, 0)
            start_gather(0)
            pltpu.make_async_copy(x_hbm.at[src_s[0]], rows_s[0],
                                  gsem[0]).wait()
            valid = jnp.where(m < n_real, 1.0, 0.0).astype(jnp.float32)
            scale_slot(0, valid)
            start_scatter(0)
            drain_scatter(0)
            return carry

        lax.fori_loop(0, npc, chunk_body, 0)
        plsc.subcore_barrier()

        @pl.when(s < n_full)
        def _():
            pltpu.sync_copy(acc.at[pl.ds(r0, rp8)],
                            out_acc.at[c, pl.ds(r0, rp8)])
        if tail_rows:
            @pl.when(s == n_full)
            def _():
                pltpu.sync_copy(acc.at[pl.ds(r0, tail_rows)],
                                out_acc.at[c, pl.ds(r0, tail_rows)])
        if compute_deg:
            pltpu.sync_copy(deg_acc.at[pl.ds(dz0, dzp)],
                            out_deg.at[pl.ds(c * npad + dz0, dzp)])

    return pl.kernel(body, out_type, mesh=mesh, scratch_types=scratch)


def _tc_mlp1(acc_ref, d0_ref, d1_ref, w_ref, b_ref, out_ref):
    x = acc_ref[0] + acc_ref[1]
    deg = jnp.maximum(d0_ref[...] + d1_ref[...], 1.0)
    h = jnp.dot(x / deg, w_ref[...], preferred_element_type=jnp.float32)
    out_ref[...] = jnp.maximum(h + b_ref[...], 0.0)


def _tc_mlp2(acc_ref, d0_ref, d1_ref, wemb_ref, bemb_ref, wcls_ref, bcls_ref,
             emb_ref, log_ref):
    x = acc_ref[0] + acc_ref[1]
    deg = jnp.maximum(d0_ref[...] + d1_ref[...], 1.0)
    e = jnp.dot(x / deg, wemb_ref[...],
                preferred_element_type=jnp.float32) + bemb_ref[...]
    emb_ref[...] = e
    log_ref[...] = jnp.dot(e, wcls_ref[...],
                           preferred_element_type=jnp.float32) + bcls_ref[...]


def kernel(node_features, edge_index, edge_weight, W_in, b_in, W_emb, b_emb,
           W_cls, b_cls):
    N, D = node_features.shape
    H = W_in.shape[1]
    EMB = W_emb.shape[1]
    C = W_cls.shape[1]
    E = edge_weight.shape[0]
    src = edge_index[0]
    dst = edge_index[1]

    acc1, deg_flat = _make_aggregate(N, D, E, True)(
        node_features, src, dst, edge_weight)
    deg = deg_flat.reshape(_NC, deg_flat.shape[0] // _NC)[:, :N]
    d0 = deg[0][:, None]
    d1 = deg[1][:, None]

    rb = 1000 if N % 1000 == 0 else N
    grid = (N // rb,)
    hidden = pl.pallas_call(
        _tc_mlp1,
        grid=grid,
        in_specs=[
            pl.BlockSpec((_NC, rb, D), lambda i: (0, i, 0)),
            pl.BlockSpec((rb, 1), lambda i: (i, 0)),
            pl.BlockSpec((rb, 1), lambda i: (i, 0)),
            pl.BlockSpec((D, H), lambda i: (0, 0)),
            pl.BlockSpec((1, H), lambda i: (0, 0)),
        ],
        out_specs=pl.BlockSpec((rb, H), lambda i: (i, 0)),
        out_shape=jax.ShapeDtypeStruct((N, H), jnp.float32),
    )(acc1, d0, d1, W_in, b_in.reshape(1, H))

    (acc2,) = _make_aggregate(N, H, E, False)(hidden, src, dst, edge_weight)

    embedding, logits = pl.pallas_call(
        _tc_mlp2,
        grid=grid,
        in_specs=[
            pl.BlockSpec((_NC, rb, H), lambda i: (0, i, 0)),
            pl.BlockSpec((rb, 1), lambda i: (i, 0)),
            pl.BlockSpec((rb, 1), lambda i: (i, 0)),
            pl.BlockSpec((H, EMB), lambda i: (0, 0)),
            pl.BlockSpec((1, EMB), lambda i: (0, 0)),
            pl.BlockSpec((EMB, C), lambda i: (0, 0)),
            pl.BlockSpec((1, C), lambda i: (0, 0)),
        ],
        out_specs=[
            pl.BlockSpec((rb, EMB), lambda i: (i, 0)),
            pl.BlockSpec((rb, C), lambda i: (i, 0)),
        ],
        out_shape=[
            jax.ShapeDtypeStruct((N, EMB), jnp.float32),
            jax.ShapeDtypeStruct((N, C), jnp.float32),
        ],
    )(acc2, d0, d1, W_emb, b_emb.reshape(1, EMB), W_cls, b_cls.reshape(1, C))

    return (embedding, logits)


# speedup vs baseline: 8.7646x; 1.6291x over previous
import functools

import jax
import jax.numpy as jnp
from jax import lax
from jax.experimental import pallas as pl
from jax.experimental.pallas import tpu as pltpu
from jax.experimental.pallas import tpu_sc as plsc

_NC = 2
_NS = 16
_NW = _NC * _NS
_CH = 128
_LANES = 16


def _make_aggregate(N, D, E, compute_deg):
    assert E % _CH == 0 and D % _LANES == 0 and N % _NS == 0
    n_chunks = E // _CH
    nb, rem = divmod(n_chunks, _NW)
    rp8 = ((N + _NS - 1) // _NS + 7) // 8 * 8
    n_full = N // rp8
    tail_rows = N - n_full * rp8
    dzp = (((N + _NS - 1) // _NS) + 127) // 128 * 128
    npad = _NS * dzp

    mesh = plsc.VectorSubcoreMesh(core_axis_name="c", subcore_axis_name="s")

    out_type = [jax.ShapeDtypeStruct((_NC, N, D), jnp.float32)]
    if compute_deg:
        out_type.append(jax.ShapeDtypeStruct((_NC * npad,), jnp.float32))

    nbmax = nb + (1 if rem else 0)
    npc = 3 * ((nbmax + 2) // 3)

    scratch = (
        [pltpu.VMEM((_CH,), jnp.int32) for _ in range(3)]
        + [pltpu.VMEM((_CH,), jnp.int32) for _ in range(3)]
        + [pltpu.VMEM((_CH,), jnp.float32) for _ in range(3)]
        + [pltpu.VMEM((_CH, D), jnp.float32) for _ in range(3)]
        + [
            pltpu.VMEM_SHARED((N, D), jnp.float32),
            pltpu.VMEM_SHARED((npad,), jnp.float32),
        ]
        + [pltpu.SemaphoreType.DMA for _ in range(9)]
    )

    def body(x_hbm, src_hbm, dst_hbm, w_hbm, *refs):
        if compute_deg:
            out_acc, out_deg = refs[0], refs[1]
            refs = refs[2:]
        else:
            out_acc = refs[0]
            refs = refs[1:]
        src_s = refs[0:3]
        dst_s = refs[3:6]
        w_s = refs[6:9]
        rows_s = refs[9:12]
        acc, deg_acc = refs[12], refs[13]
        gsem = refs[14:17]
        ssem = refs[17:20]
        dsem = refs[20:23]
        rows_v = rows_s[0]
        w_v = w_s[0]

        c = lax.axis_index("c")
        s = lax.axis_index("s")
        wid = s * _NC + c

        def zrow(r, carry):
            for cb in range(D // _LANES):
                rows_v[r, pl.ds(cb * _LANES, _LANES)] = jnp.zeros(
                    (_LANES,), jnp.float32)
            return carry
        lax.fori_loop(0, _CH, zrow, 0)
        for g in range(_CH // _LANES):
            w_v[pl.ds(g * _LANES, _LANES)] = jnp.zeros((_LANES,), jnp.float32)

        r0 = s * rp8

        def _zero_rows(nrows):
            full, tail = divmod(nrows, _CH)
            for i in range(full):
                pltpu.sync_copy(rows_v, acc.at[pl.ds(r0 + i * _CH, _CH)])
            if tail:
                pltpu.sync_copy(rows_v.at[pl.ds(0, tail)],
                                acc.at[pl.ds(r0 + full * _CH, tail)])

        @pl.when(s < n_full)
        def _():
            _zero_rows(rp8)
        if tail_rows:
            @pl.when(s == n_full)
            def _():
                _zero_rows(tail_rows)

        dz0 = s * dzp
        dfull, dtail = divmod(dzp, _CH)
        for i in range(dfull):
            pltpu.sync_copy(w_v, deg_acc.at[pl.ds(dz0 + i * _CH, _CH)])
        if dtail:
            pltpu.sync_copy(w_v.at[pl.ds(0, dtail)],
                            deg_acc.at[pl.ds(dz0 + dfull * _CH, dtail)])
        plsc.subcore_barrier()

        n_real = nb + jnp.where(wid < rem, 1, 0)

        def chunk_base(m):
            return (wid + lax.rem(m, n_real) * _NW) * _CH

        def load_idx(m, s):
            base = chunk_base(m)
            pltpu.sync_copy(src_hbm.at[pl.ds(base, _CH)], src_s[s])
            pltpu.sync_copy(dst_hbm.at[pl.ds(base, _CH)], dst_s[s])
            pltpu.sync_copy(w_hbm.at[pl.ds(base, _CH)], w_s[s])

        def start_gather(s):
            pltpu.async_copy(x_hbm.at[src_s[s]], rows_s[s], gsem[s])

        def scale_slot(s, valid):
            def scale(g, cc):
                w_grp = w_s[s][pl.ds(g * _LANES, _LANES)] * valid
                w_s[s][pl.ds(g * _LANES, _LANES)] = w_grp
                for j in range(_LANES):
                    wr = w_grp[j]
                    r = g * _LANES + j
                    for cb in range(D // _LANES):
                        sl = pl.ds(cb * _LANES, _LANES)
                        rows_s[s][r, sl] = rows_s[s][r, sl] * wr
                return cc
            lax.fori_loop(0, _CH // _LANES, scale, 0)

        def start_scatter(s):
            pltpu.async_copy(rows_s[s], acc.at[dst_s[s]], ssem[s], add=True)
            if compute_deg:
                pltpu.async_copy(w_s[s], deg_acc.at[dst_s[s]], dsem[s],
                                 add=True)

        def drain_scatter(s):
            pltpu.make_async_copy(rows_s[s], acc.at[dst_s[s]], ssem[s]).wait()
            if compute_deg:
                pltpu.make_async_copy(w_s[s], deg_acc.at[dst_s[s]],
                                      dsem[s]).wait()

        def wait_gather(s):
            pltpu.make_async_copy(x_hbm.at[src_s[s]], rows_s[s],
                                  gsem[s]).wait()

        load_idx(jnp.int32(0), 0)
        start_gather(0)
        load_idx(jnp.int32(1), 1)
        start_gather(1)

        def zrow2(r, carry):
            for cb in range(D // _LANES):
                rows_s[2][r, pl.ds(cb * _LANES, _LANES)] = jnp.zeros(
                    (_LANES,), jnp.float32)
            return carry
        lax.fori_loop(0, _CH, zrow2, 0)
        for g in range(_CH // _LANES):
            w_s[2][pl.ds(g * _LANES, _LANES)] = jnp.zeros(
                (_LANES,), jnp.float32)
            dst_s[2][pl.ds(g * _LANES, _LANES)] = jnp.zeros(
                (_LANES,), jnp.int32)
        start_scatter(2)

        def rot_body(p, carry):
            for j in range(3):
                m = 3 * p + j
                wait_gather(j)
                valid = jnp.where(m < n_real, 1.0, 0.0).astype(jnp.float32)
                scale_slot(j, valid)
                start_scatter(j)
                y = (j + 2) % 3
                drain_scatter(y)
                load_idx(m + 2, y)
                start_gather(y)
            return carry

        lax.fori_loop(0, npc // 3, rot_body, 0)
        drain_scatter(2)
        wait_gather(0)
        wait_gather(1)
        plsc.subcore_barrier()

        @pl.when(s < n_full)
        def _():
            pltpu.sync_copy(acc.at[pl.ds(r0, rp8)],
                            out_acc.at[c, pl.ds(r0, rp8)])
        if tail_rows:
            @pl.when(s == n_full)
            def _():
                pltpu.sync_copy(acc.at[pl.ds(r0, tail_rows)],
                                out_acc.at[c, pl.ds(r0, tail_rows)])
        if compute_deg:
            pltpu.sync_copy(deg_acc.at[pl.ds(dz0, dzp)],
                            out_deg.at[pl.ds(c * npad + dz0, dzp)])

    return pl.kernel(body, out_type, mesh=mesh, scratch_types=scratch)


def _tc_mlp1(acc_ref, d0_ref, d1_ref, w_ref, b_ref, out_ref):
    x = acc_ref[0] + acc_ref[1]
    deg = jnp.maximum(d0_ref[...] + d1_ref[...], 1.0)
    h = jnp.dot(x / deg, w_ref[...], preferred_element_type=jnp.float32)
    out_ref[...] = jnp.maximum(h + b_ref[...], 0.0)


def _tc_mlp2(acc_ref, d0_ref, d1_ref, wemb_ref, bemb_ref, wcls_ref, bcls_ref,
             emb_ref, log_ref):
    x = acc_ref[0] + acc_ref[1]
    deg = jnp.maximum(d0_ref[...] + d1_ref[...], 1.0)
    e = jnp.dot(x / deg, wemb_ref[...],
                preferred_element_type=jnp.float32) + bemb_ref[...]
    emb_ref[...] = e
    log_ref[...] = jnp.dot(e, wcls_ref[...],
                           preferred_element_type=jnp.float32) + bcls_ref[...]


def kernel(node_features, edge_index, edge_weight, W_in, b_in, W_emb, b_emb,
           W_cls, b_cls):
    N, D = node_features.shape
    H = W_in.shape[1]
    EMB = W_emb.shape[1]
    C = W_cls.shape[1]
    E = edge_weight.shape[0]
    src = edge_index[0]
    dst = edge_index[1]

    acc1, deg_flat = _make_aggregate(N, D, E, True)(
        node_features, src, dst, edge_weight)
    deg = deg_flat.reshape(_NC, deg_flat.shape[0] // _NC)[:, :N]
    d0 = deg[0][:, None]
    d1 = deg[1][:, None]

    rb = 1000 if N % 1000 == 0 else N
    grid = (N // rb,)
    hidden = pl.pallas_call(
        _tc_mlp1,
        grid=grid,
        in_specs=[
            pl.BlockSpec((_NC, rb, D), lambda i: (0, i, 0)),
            pl.BlockSpec((rb, 1), lambda i: (i, 0)),
            pl.BlockSpec((rb, 1), lambda i: (i, 0)),
            pl.BlockSpec((D, H), lambda i: (0, 0)),
            pl.BlockSpec((1, H), lambda i: (0, 0)),
        ],
        out_specs=pl.BlockSpec((rb, H), lambda i: (i, 0)),
        out_shape=jax.ShapeDtypeStruct((N, H), jnp.float32),
    )(acc1, d0, d1, W_in, b_in.reshape(1, H))

    (acc2,) = _make_aggregate(N, H, E, False)(hidden, src, dst, edge_weight)

    embedding, logits = pl.pallas_call(
        _tc_mlp2,
        grid=grid,
        in_specs=[
            pl.BlockSpec((_NC, rb, H), lambda i: (0, i, 0)),
            pl.BlockSpec((rb, 1), lambda i: (i, 0)),
            pl.BlockSpec((rb, 1), lambda i: (i, 0)),
            pl.BlockSpec((H, EMB), lambda i: (0, 0)),
            pl.BlockSpec((1, EMB), lambda i: (0, 0)),
            pl.BlockSpec((EMB, C), lambda i: (0, 0)),
            pl.BlockSpec((1, C), lambda i: (0, 0)),
        ],
        out_specs=[
            pl.BlockSpec((rb, EMB), lambda i: (i, 0)),
            pl.BlockSpec((rb, C), lambda i: (i, 0)),
        ],
        out_shape=[
            jax.ShapeDtypeStruct((N, EMB), jnp.float32),
            jax.ShapeDtypeStruct((N, C), jnp.float32),
        ],
    )(acc2, d0, d1, W_emb, b_emb.reshape(1, EMB), W_cls, b_cls.reshape(1, C))

    return (embedding, logits)


# speedup vs baseline: 11.7908x; 1.3453x over previous
import functools

import jax
import jax.numpy as jnp
from jax import lax
from jax.experimental import pallas as pl
from jax.experimental.pallas import tpu as pltpu
from jax.experimental.pallas import tpu_sc as plsc

_NC = 2
_NS = 16
_NW = _NC * _NS
_CH = 128
_LANES = 16


def _make_aggregate(N, D, E, compute_deg):
    assert E % _CH == 0 and D % _LANES == 0 and N % _NS == 0
    n_chunks = E // _CH
    nb, rem = divmod(n_chunks, _NW)
    rp8 = ((N + _NS - 1) // _NS + 7) // 8 * 8
    n_full = N // rp8
    tail_rows = N - n_full * rp8
    dzp = (((N + _NS - 1) // _NS) + 127) // 128 * 128
    npad = _NS * dzp

    mesh = plsc.VectorSubcoreMesh(core_axis_name="c", subcore_axis_name="s")

    out_type = [jax.ShapeDtypeStruct((_NC, N, D), jnp.float32)]
    if compute_deg:
        out_type.append(jax.ShapeDtypeStruct((_NC * npad,), jnp.float32))

    nbmax = nb + (1 if rem else 0)
    npc = 3 * ((nbmax + 2) // 3)

    scratch = (
        [pltpu.VMEM((_CH,), jnp.int32) for _ in range(3)]
        + [pltpu.VMEM((_CH,), jnp.int32) for _ in range(3)]
        + [pltpu.VMEM((_CH,), jnp.float32) for _ in range(3)]
        + [pltpu.VMEM((_CH, D), jnp.float32) for _ in range(3)]
        + [
            pltpu.VMEM_SHARED((N, D), jnp.float32),
            pltpu.VMEM_SHARED((npad,), jnp.float32),
        ]
        + [pltpu.SemaphoreType.DMA for _ in range(12)]
    )

    def body(x_hbm, src_hbm, dst_hbm, w_hbm, *refs):
        if compute_deg:
            out_acc, out_deg = refs[0], refs[1]
            refs = refs[2:]
        else:
            out_acc = refs[0]
            refs = refs[1:]
        src_s = refs[0:3]
        dst_s = refs[3:6]
        w_s = refs[6:9]
        rows_s = refs[9:12]
        acc, deg_acc = refs[12], refs[13]
        gsem = refs[14:17]
        ssem = refs[17:20]
        dsem = refs[20:23]
        isem = refs[23:26]
        rows_v = rows_s[0]
        w_v = w_s[0]

        c = lax.axis_index("c")
        s = lax.axis_index("s")
        wid = s * _NC + c

        def zrow(r, carry):
            for cb in range(D // _LANES):
                rows_v[r, pl.ds(cb * _LANES, _LANES)] = jnp.zeros(
                    (_LANES,), jnp.float32)
            return carry
        lax.fori_loop(0, _CH, zrow, 0)
        for g in range(_CH // _LANES):
            w_v[pl.ds(g * _LANES, _LANES)] = jnp.zeros((_LANES,), jnp.float32)

        r0 = s * rp8

        def _zero_rows(nrows):
            full, tail = divmod(nrows, _CH)
            for i in range(full):
                pltpu.sync_copy(rows_v, acc.at[pl.ds(r0 + i * _CH, _CH)])
            if tail:
                pltpu.sync_copy(rows_v.at[pl.ds(0, tail)],
                                acc.at[pl.ds(r0 + full * _CH, tail)])

        @pl.when(s < n_full)
        def _():
            _zero_rows(rp8)
        if tail_rows:
            @pl.when(s == n_full)
            def _():
                _zero_rows(tail_rows)

        dz0 = s * dzp
        dfull, dtail = divmod(dzp, _CH)
        for i in range(dfull):
            pltpu.sync_copy(w_v, deg_acc.at[pl.ds(dz0 + i * _CH, _CH)])
        if dtail:
            pltpu.sync_copy(w_v.at[pl.ds(0, dtail)],
                            deg_acc.at[pl.ds(dz0 + dfull * _CH, dtail)])
        plsc.subcore_barrier()

        n_real = nb + jnp.where(wid < rem, 1, 0)

        def chunk_base(m):
            return (wid + lax.rem(m, n_real) * _NW) * _CH

        def load_idx(m, s):
            base = chunk_base(m)
            pltpu.sync_copy(src_hbm.at[pl.ds(base, _CH)], src_s[s])
            pltpu.sync_copy(dst_hbm.at[pl.ds(base, _CH)], dst_s[s])
            pltpu.sync_copy(w_hbm.at[pl.ds(base, _CH)], w_s[s])

        def start_gather(s):
            pltpu.async_copy(x_hbm.at[src_s[s]], rows_s[s], gsem[s])

        def scale_slot(s, valid, g_lo, g_hi):
            def scale(g, cc):
                w_grp = w_s[s][pl.ds(g * _LANES, _LANES)] * valid
                if compute_deg:
                    w_s[s][pl.ds(g * _LANES, _LANES)] = w_grp
                for j in range(_LANES):
                    wr = w_grp[j]
                    r = g * _LANES + j
                    for cb in range(D // _LANES):
                        sl = pl.ds(cb * _LANES, _LANES)
                        rows_s[s][r, sl] = rows_s[s][r, sl] * wr
                return cc
            lax.fori_loop(g_lo, g_hi, scale, 0)

        def start_idx(m, s):
            base = chunk_base(m)
            pltpu.async_copy(src_hbm.at[pl.ds(base, _CH)], src_s[s], isem[s])
            pltpu.async_copy(dst_hbm.at[pl.ds(base, _CH)], dst_s[s], isem[s])
            pltpu.async_copy(w_hbm.at[pl.ds(base, _CH)], w_s[s], isem[s])

        def wait_idx(s):
            z = pl.ds(0, _CH)
            pltpu.make_async_copy(src_hbm.at[z], src_s[s], isem[s]).wait()
            pltpu.make_async_copy(dst_hbm.at[z], dst_s[s], isem[s]).wait()
            pltpu.make_async_copy(w_hbm.at[z], w_s[s], isem[s]).wait()

        def start_scatter(s):
            pltpu.async_copy(rows_s[s], acc.at[dst_s[s]], ssem[s], add=True)
            if compute_deg:
                pltpu.async_copy(w_s[s], deg_acc.at[dst_s[s]], dsem[s],
                                 add=True)

        def drain_scatter(s):
            pltpu.make_async_copy(rows_s[s], acc.at[dst_s[s]], ssem[s]).wait()
            if compute_deg:
                pltpu.make_async_copy(w_s[s], deg_acc.at[dst_s[s]],
                                      dsem[s]).wait()

        def wait_gather(s):
            pltpu.make_async_copy(x_hbm.at[src_s[s]], rows_s[s],
                                  gsem[s]).wait()

        load_idx(jnp.int32(0), 0)
        start_gather(0)
        load_idx(jnp.int32(1), 1)
        start_gather(1)

        def zrow2(r, carry):
            for cb in range(D // _LANES):
                rows_s[2][r, pl.ds(cb * _LANES, _LANES)] = jnp.zeros(
                    (_LANES,), jnp.float32)
            return carry
        lax.fori_loop(0, _CH, zrow2, 0)
        for g in range(_CH // _LANES):
            w_s[2][pl.ds(g * _LANES, _LANES)] = jnp.zeros(
                (_LANES,), jnp.float32)
            dst_s[2][pl.ds(g * _LANES, _LANES)] = jnp.zeros(
                (_LANES,), jnp.int32)
        start_scatter(2)

        n_grp = _CH // _LANES
        g_split = 5 * n_grp // 8

        def rot_body(p, carry):
            for j in range(3):
                m = 3 * p + j
                wait_gather(j)
                valid = jnp.where(m < n_real, 1.0, 0.0).astype(jnp.float32)
                scale_slot(j, valid, 0, g_split)
                y = (j + 2) % 3
                drain_scatter(y)
                start_idx(m + 2, y)
                scale_slot(j, valid, g_split, n_grp)
                start_scatter(j)
                wait_idx(y)
                start_gather(y)
            return carry

        lax.fori_loop(0, npc // 3, rot_body, 0)
        drain_scatter(2)
        wait_gather(0)
        wait_gather(1)
        plsc.subcore_barrier()

        @pl.when(s < n_full)
        def _():
            pltpu.sync_copy(acc.at[pl.ds(r0, rp8)],
                            out_acc.at[c, pl.ds(r0, rp8)])
        if tail_rows:
            @pl.when(s == n_full)
            def _():
                pltpu.sync_copy(acc.at[pl.ds(r0, tail_rows)],
                                out_acc.at[c, pl.ds(r0, tail_rows)])
        if compute_deg:
            pltpu.sync_copy(deg_acc.at[pl.ds(dz0, dzp)],
                            out_deg.at[pl.ds(c * npad + dz0, dzp)])

    return pl.kernel(body, out_type, mesh=mesh, scratch_types=scratch)


def _tc_mlp1(acc_ref, d0_ref, d1_ref, w_ref, b_ref, out_ref):
    x = acc_ref[0] + acc_ref[1]
    deg = jnp.maximum(d0_ref[...] + d1_ref[...], 1.0)
    h = jnp.dot(x / deg, w_ref[...], preferred_element_type=jnp.float32)
    out_ref[...] = jnp.maximum(h + b_ref[...], 0.0)


def _tc_mlp2(acc_ref, d0_ref, d1_ref, wemb_ref, bemb_ref, wcls_ref, bcls_ref,
             emb_ref, log_ref):
    x = acc_ref[0] + acc_ref[1]
    deg = jnp.maximum(d0_ref[...] + d1_ref[...], 1.0)
    e = jnp.dot(x / deg, wemb_ref[...],
                preferred_element_type=jnp.float32) + bemb_ref[...]
    emb_ref[...] = e
    log_ref[...] = jnp.dot(e, wcls_ref[...],
                           preferred_element_type=jnp.float32) + bcls_ref[...]


def kernel(node_features, edge_index, edge_weight, W_in, b_in, W_emb, b_emb,
           W_cls, b_cls):
    N, D = node_features.shape
    H = W_in.shape[1]
    EMB = W_emb.shape[1]
    C = W_cls.shape[1]
    E = edge_weight.shape[0]
    src = edge_index[0]
    dst = edge_index[1]

    acc1, deg_flat = _make_aggregate(N, D, E, True)(
        node_features, src, dst, edge_weight)
    deg = deg_flat.reshape(_NC, deg_flat.shape[0] // _NC)[:, :N]
    d0 = deg[0][:, None]
    d1 = deg[1][:, None]

    rb = 1000 if N % 1000 == 0 else N
    grid = (N // rb,)
    hidden = pl.pallas_call(
        _tc_mlp1,
        grid=grid,
        in_specs=[
            pl.BlockSpec((_NC, rb, D), lambda i: (0, i, 0)),
            pl.BlockSpec((rb, 1), lambda i: (i, 0)),
            pl.BlockSpec((rb, 1), lambda i: (i, 0)),
            pl.BlockSpec((D, H), lambda i: (0, 0)),
            pl.BlockSpec((1, H), lambda i: (0, 0)),
        ],
        out_specs=pl.BlockSpec((rb, H), lambda i: (i, 0)),
        out_shape=jax.ShapeDtypeStruct((N, H), jnp.float32),
    )(acc1, d0, d1, W_in, b_in.reshape(1, H))

    (acc2,) = _make_aggregate(N, H, E, False)(hidden, src, dst, edge_weight)

    embedding, logits = pl.pallas_call(
        _tc_mlp2,
        grid=grid,
        in_specs=[
            pl.BlockSpec((_NC, rb, H), lambda i: (0, i, 0)),
            pl.BlockSpec((rb, 1), lambda i: (i, 0)),
            pl.BlockSpec((rb, 1), lambda i: (i, 0)),
            pl.BlockSpec((H, EMB), lambda i: (0, 0)),
            pl.BlockSpec((1, EMB), lambda i: (0, 0)),
            pl.BlockSpec((EMB, C), lambda i: (0, 0)),
            pl.BlockSpec((1, C), lambda i: (0, 0)),
        ],
        out_specs=[
            pl.BlockSpec((rb, EMB), lambda i: (i, 0)),
            pl.BlockSpec((rb, C), lambda i: (i, 0)),
        ],
        out_shape=[
            jax.ShapeDtypeStruct((N, EMB), jnp.float32),
            jax.ShapeDtypeStruct((N, C), jnp.float32),
        ],
    )(acc2, d0, d1, W_emb, b_emb.reshape(1, EMB), W_cls, b_cls.reshape(1, C))

    return (embedding, logits)
